# baseline (speedup 1.0000x reference)
import functools

import jax
import jax.numpy as jnp
from jax.experimental import pallas as pl
from jax.experimental.pallas import tpu as pltpu

BB = 4
DEPTH = 2


def _se_kernel(x_hbm, w1_ref, w2_ref, o_hbm, ibuf, obuf, isems, osems,
               *, nsteps, inv_hw):
    def start_in(i, slot):
        pltpu.make_async_copy(
            x_hbm.at[pl.ds(i * BB, BB)], ibuf.at[slot], isems.at[slot]).start()

    def wait_in(i, slot):
        pltpu.make_async_copy(
            x_hbm.at[pl.ds(i * BB, BB)], ibuf.at[slot], isems.at[slot]).wait()

    def start_out(i, slot):
        pltpu.make_async_copy(
            obuf.at[slot], o_hbm.at[pl.ds(i * BB, BB)], osems.at[slot]).start()

    def wait_out(i, slot):
        pltpu.make_async_copy(
            obuf.at[slot], o_hbm.at[pl.ds(i * BB, BB)], osems.at[slot]).wait()

    for s in range(DEPTH):
        start_in(jnp.int32(s), s)

    def loop(i, carry):
        slot = jax.lax.rem(i, DEPTH)
        wait_in(i, slot)

        x = ibuf[slot]
        mean = jnp.sum(x, axis=-1, dtype=jnp.float32) * inv_hw
        hidden = jnp.maximum(
            jnp.dot(mean, w1_ref[...], preferred_element_type=jnp.float32),
            0.0)
        gate = jax.nn.sigmoid(
            jnp.dot(hidden, w2_ref[...], preferred_element_type=jnp.float32))

        @pl.when(i >= DEPTH)
        def _():
            wait_out(i - DEPTH, slot)

        obuf[slot] = x * gate[:, :, None].astype(x.dtype)
        start_out(i, slot)

        @pl.when(i + DEPTH < nsteps)
        def _():
            start_in(i + DEPTH, slot)
        return carry

    jax.lax.fori_loop(0, nsteps, loop, 0)
    for s in range(DEPTH):
        wait_out(nsteps - DEPTH + s, jax.lax.rem(jnp.int32(nsteps - DEPTH + s), DEPTH))


def kernel(x_nchw, w1, w2):
    B, C, H, W = x_nchw.shape
    HW = H * W
    x = x_nchw.reshape(B, C, HW)
    nsteps = B // BB

    out = pl.pallas_call(
        functools.partial(_se_kernel, nsteps=nsteps, inv_hw=1.0 / HW),
        out_shape=jax.ShapeDtypeStruct((B, C, HW), x.dtype),
        in_specs=[
            pl.BlockSpec(memory_space=pltpu.HBM),
            pl.BlockSpec(memory_space=pltpu.VMEM),
            pl.BlockSpec(memory_space=pltpu.VMEM),
        ],
        out_specs=pl.BlockSpec(memory_space=pltpu.HBM),
        scratch_shapes=[
            pltpu.VMEM((DEPTH, BB, C, HW), jnp.float32),
            pltpu.VMEM((DEPTH, BB, C, HW), jnp.float32),
            pltpu.SemaphoreType.DMA((DEPTH,)),
            pltpu.SemaphoreType.DMA((DEPTH,)),
        ],
        compiler_params=pltpu.CompilerParams(vmem_limit_bytes=64 << 20),
    )(x, w1.astype(jnp.float32), w2.astype(jnp.float32))

    return out.reshape(B, C, H, W)


# speedup vs baseline: 1.3267x; 1.3267x over previous
import functools

import jax
import jax.numpy as jnp
from jax.experimental import pallas as pl
from jax.experimental.pallas import tpu as pltpu


def _gate_kernel(x_ref, w1_ref, w2_ref, g_ref, acc, *, inv_hw):
    h = pl.program_id(1)

    @pl.when(h == 0)
    def _():
        acc[...] = jnp.zeros_like(acc)

    acc[...] += jnp.sum(x_ref[...], axis=-1, dtype=jnp.float32)

    @pl.when(h == pl.num_programs(1) - 1)
    def _():
        mean = acc[...] * inv_hw
        hidden = jnp.maximum(
            jnp.dot(mean, w1_ref[...], preferred_element_type=jnp.float32),
            0.0)
        g_ref[...] = jax.nn.sigmoid(
            jnp.dot(hidden, w2_ref[...], preferred_element_type=jnp.float32))


def kernel(x_nchw, w1, w2):
    B, C, H, W = x_nchw.shape
    Cr = w1.shape[1]
    HW = H * W
    x = x_nchw.reshape(B, C, HW)

    bB, bHW = 8, HW
    grid = (B // bB, HW // bHW)
    gate = pl.pallas_call(
        functools.partial(_gate_kernel, inv_hw=1.0 / HW),
        out_shape=jax.ShapeDtypeStruct((B, C), jnp.float32),
        grid=grid,
        in_specs=[
            pl.BlockSpec((bB, C, bHW), lambda b, h: (b, 0, h)),
            pl.BlockSpec((C, Cr), lambda b, h: (0, 0)),
            pl.BlockSpec((Cr, C), lambda b, h: (0, 0)),
        ],
        out_specs=pl.BlockSpec((bB, C), lambda b, h: (b, 0)),
        scratch_shapes=[pltpu.VMEM((bB, C), jnp.float32)],
        compiler_params=pltpu.CompilerParams(
            dimension_semantics=("parallel", "arbitrary"),
            vmem_limit_bytes=64 << 20),
        cost_estimate=pl.CostEstimate(
            flops=B * C * HW + B * 2 * C * Cr * 2,
            transcendentals=B * C,
            bytes_accessed=B * C * HW * 4 + B * C * 4),
    )(x, w1.astype(jnp.float32), w2.astype(jnp.float32))

    out = x_nchw * gate[:, :, None, None].astype(x_nchw.dtype)
    return out
